# baseline (speedup 1.0000x reference)
import jax
import jax.numpy as jnp
from jax import lax
from jax.experimental import pallas as pl
from jax.experimental.pallas import tpu as pltpu
from jax.experimental.pallas import tpu_sc as plsc

D = 16
L = 200
B = 16384
NC, NS = 2, 16
NW = NC * NS
BAGS_PER_W = B // NW
G = 16
NG = BAGS_PER_W // G
IDX_PER_G = G * L
BLK = 128
NBLK = IDX_PER_G // BLK


def _fire_group(text_hbm, table_hbm, idx_v, rows_v, sem, base):
    pltpu.sync_copy(text_hbm.at[pl.ds(base, IDX_PER_G)], idx_v)
    for j in range(NBLK):
        pltpu.async_copy(
            table_hbm.at[idx_v.at[pl.ds(j * BLK, BLK)]],
            rows_v.at[pl.ds(j * BLK, BLK)],
            sem,
        )


def _drain_group(table_hbm, rows_v, sem):
    pltpu.make_async_copy(table_hbm.at[pl.ds(0, IDX_PER_G)], rows_v, sem).wait()


def _process_group(rows_v, wt_rows, b_vec, out_v, out_hbm, obase):
    def bag_body(bag, carry):
        rbase = bag * L
        z = jnp.zeros((D,), jnp.float32)

        @plsc.parallel_loop(0, L // 8, unroll=5, carry=(z, z, z, z))
        def accs(i, accs):
            a0, a1, a2, a3 = accs
            r = rbase + i * 8
            a0 = a0 + (rows_v[r, :] + rows_v[r + 4, :])
            a1 = a1 + (rows_v[r + 1, :] + rows_v[r + 5, :])
            a2 = a2 + (rows_v[r + 2, :] + rows_v[r + 6, :])
            a3 = a3 + (rows_v[r + 3, :] + rows_v[r + 7, :])
            return a0, a1, a2, a3

        a0, a1, a2, a3 = accs
        s = (a0 + a1) + (a2 + a3)

        parts = [b_vec, z, z, z]
        for k in range(D):
            parts[k % 4] = parts[k % 4] + s[k] * wt_rows[k]
        out_v[bag, :] = (parts[0] + parts[1]) + (parts[2] + parts[3])
        return carry

    lax.fori_loop(0, G, bag_body, 0)
    pltpu.sync_copy(out_v, out_hbm.at[pl.ds(obase, G), :])


def _body(text_hbm, table_hbm, wt_hbm, bias_hbm, out_hbm,
          idx0, idx1, rows0, rows1, wt_v, b_v, out_v, sem0, sem1):
    wid = lax.axis_index("s") * NC + lax.axis_index("c")
    tbase = wid * BAGS_PER_W * L
    obase = wid * BAGS_PER_W

    pltpu.sync_copy(wt_hbm, wt_v)
    pltpu.sync_copy(bias_hbm, b_v)
    wt_rows = [wt_v[k, :] for k in range(D)]
    b_vec = b_v[:]

    _fire_group(text_hbm, table_hbm, idx0, rows0, sem0, tbase)

    def outer(g2, carry):
        gA = g2 * 2
        gB = gA + 1
        _fire_group(text_hbm, table_hbm, idx1, rows1, sem1,
                    tbase + gB * IDX_PER_G)
        _drain_group(table_hbm, rows0, sem0)
        _process_group(rows0, wt_rows, b_vec, out_v, out_hbm, obase + gA * G)

        @pl.when(g2 < NG // 2 - 1)
        def _():
            _fire_group(text_hbm, table_hbm, idx0, rows0, sem0,
                        tbase + (gA + 2) * IDX_PER_G)

        _drain_group(table_hbm, rows1, sem1)
        _process_group(rows1, wt_rows, b_vec, out_v, out_hbm, obase + gB * G)
        return carry

    lax.fori_loop(0, NG // 2, outer, 0)


V = 1000000
NTC_FULL = V // 128
BT = 4
NB_MAIN = 61
TAIL_B = NB_MAIN * NW
TAIL_COLS = V - NTC_FULL * 128


def _detile_cols(tt_hbm, bufs, semi, b):
    for p in range(BT):
        pltpu.async_copy(
            tt_hbm.at[:, pl.ds((b * BT + p) * 128, 128)],
            bufs[p].at[:, pl.ds(0, 128)], semi)


def _detile_wait(tt_hbm, bufs, semi, b):
    for p in range(BT):
        pltpu.make_async_copy(
            tt_hbm.at[:, pl.ds((b * BT + p) * 128, 128)],
            bufs[p].at[:, pl.ds(0, 128)], semi,
        ).wait()


def _detile_emit(bufs, ob, ncols=128):
    iota16 = lax.iota(jnp.int32, 16)

    for p in range(BT):
        @plsc.parallel_loop(0, ncols, unroll=32)
        def _(v, p=p):
            col = plsc.load_gather(bufs[p], [iota16, jnp.full((16,), v, jnp.int32)])
            ob[pl.ds(p * 2048 + v * D, D)] = col


def _detile_body(tt_hbm, tail_hbm, out_hbm, inb, obb,
                 semi0, semi1, semi2, semi3, semo0, semo1, semo2, semo3):
    wid = lax.axis_index("s") * NC + lax.axis_index("c")
    base = wid * NB_MAIN
    semi = [semi0, semi1, semi2, semi3]
    semo = [semo0, semo1, semo2, semo3]
    slot_bufs = [[inb.at[pl.ds((s * BT + p) * 16, 16), :] for p in range(BT)]
                 for s in range(4)]
    obs = [obb.at[pl.ds(s * BT * 2048, BT * 2048)] for s in range(4)]

    for s in range(4):
        _detile_cols(tt_hbm, slot_bufs[s], semi[s], base + s)

    def ring(i, carry):
        for s in range(4):
            b = base + 4 * i + s
            _detile_wait(tt_hbm, slot_bufs[s], semi[s], b)

            @pl.when(i > 0)
            def _(s=s):
                pltpu.make_async_copy(
                    obs[s], out_hbm.at[pl.ds(0, BT * 2048)], semo[s]).wait()

            _detile_emit(slot_bufs[s], obs[s])
            pltpu.async_copy(
                obs[s], out_hbm.at[pl.ds(b * BT * 2048, BT * 2048)], semo[s])
            if s == 0:
                _detile_cols(tt_hbm, slot_bufs[s], semi[s], b + 4)
            else:
                @pl.when(i < 14)
                def _(s=s, b=b):
                    _detile_cols(tt_hbm, slot_bufs[s], semi[s], b + 4)
        return carry

    lax.fori_loop(0, 15, ring, 0)

    bL = base + NB_MAIN - 1
    _detile_wait(tt_hbm, slot_bufs[0], semi[0], bL)
    pltpu.make_async_copy(obs[0], out_hbm.at[pl.ds(0, BT * 2048)], semo[0]).wait()
    _detile_emit(slot_bufs[0], obs[0])
    pltpu.async_copy(obs[0], out_hbm.at[pl.ds(bL * BT * 2048, BT * 2048)], semo[0])
    for s in range(4):
        pltpu.make_async_copy(
            obs[s], out_hbm.at[pl.ds(0, BT * 2048)], semo[s]).wait()

    @pl.when(wid == 0)
    def _():
        _detile_cols(tt_hbm, slot_bufs[0], semi[0], TAIL_B)
        _detile_wait(tt_hbm, slot_bufs[0], semi[0], TAIL_B)
        _detile_emit(slot_bufs[0], obs[0])
        pltpu.sync_copy(obs[0], out_hbm.at[pl.ds(TAIL_B * BT * 2048, BT * 2048)])

    @pl.when(wid == 1)
    def _():
        pltpu.sync_copy(tail_hbm, obs[1].at[pl.ds(0, TAIL_COLS * D)])
        pltpu.sync_copy(obs[1].at[pl.ds(0, TAIL_COLS * D)],
                        out_hbm.at[pl.ds(NTC_FULL * 128 * D, TAIL_COLS * D)])


def _detile(table_t, tail_flat):
    run = pl.kernel(
        _detile_body,
        out_type=jax.ShapeDtypeStruct((V * D,), jnp.float32),
        mesh=plsc.VectorSubcoreMesh(core_axis_name="c", subcore_axis_name="s"),
        compiler_params=pltpu.CompilerParams(use_tc_tiling_on_sc=True,
                                             needs_layout_passes=False),
        scratch_types=(
            [pltpu.VMEM((4 * BT * 16, 129), jnp.float32),
             pltpu.VMEM((4 * BT * 2048,), jnp.float32)]
            + [pltpu.SemaphoreType.DMA for _ in range(8)]
        ),
    )
    return run(table_t, tail_flat).reshape(V, D)


@jax.jit
def kernel(text, table, W, b):
    text_flat = text.reshape(-1).astype(jnp.int32)
    table_lin = _detile(table.T, table[NTC_FULL * 128:, :].reshape(-1))
    wt = (W.T / jnp.float32(L)).astype(jnp.float32)
    run = pl.kernel(
        _body,
        out_type=jax.ShapeDtypeStruct((B, D), jnp.float32),
        mesh=plsc.VectorSubcoreMesh(core_axis_name="c", subcore_axis_name="s"),
        compiler_params=pltpu.CompilerParams(use_tc_tiling_on_sc=False),
        scratch_types=[
            pltpu.VMEM((IDX_PER_G,), jnp.int32),
            pltpu.VMEM((IDX_PER_G,), jnp.int32),
            pltpu.VMEM((IDX_PER_G, D), jnp.float32),
            pltpu.VMEM((IDX_PER_G, D), jnp.float32),
            pltpu.VMEM((D, D), jnp.float32),
            pltpu.VMEM((D,), jnp.float32),
            pltpu.VMEM((G, D), jnp.float32),
            pltpu.SemaphoreType.DMA,
            pltpu.SemaphoreType.DMA,
        ],
    )
    return run(text_flat, table_lin, wt, b.astype(jnp.float32))


# speedup vs baseline: 1.0136x; 1.0136x over previous
import jax
import jax.numpy as jnp
from jax import lax
from jax.experimental import pallas as pl
from jax.experimental.pallas import tpu as pltpu
from jax.experimental.pallas import tpu_sc as plsc

D = 16
L = 200
B = 16384
NC, NS = 2, 16
NW = NC * NS
BAGS_PER_W = B // NW
G = 16
NG = BAGS_PER_W // G
IDX_PER_G = G * L
BLK = 128
NBLK = IDX_PER_G // BLK


def _fire_group(text_hbm, table_hbm, idx_v, rows_v, sem, base):
    pltpu.sync_copy(text_hbm.at[pl.ds(base, IDX_PER_G)], idx_v)
    for j in range(NBLK):
        pltpu.async_copy(
            table_hbm.at[idx_v.at[pl.ds(j * BLK, BLK)]],
            rows_v.at[pl.ds(j * BLK, BLK)],
            sem,
        )


def _drain_group(table_hbm, rows_v, sem):
    pltpu.make_async_copy(table_hbm.at[pl.ds(0, IDX_PER_G)], rows_v, sem).wait()


def _process_group(rows_v, wt_rows, b_vec, out_v, out_hbm, obase):
    def bag_body(bag, carry):
        rbase = bag * L
        z = jnp.zeros((D,), jnp.float32)

        @plsc.parallel_loop(0, L // 8, unroll=5, carry=(z, z, z, z))
        def accs(i, accs):
            a0, a1, a2, a3 = accs
            r = rbase + i * 8
            a0 = a0 + (rows_v[r, :] + rows_v[r + 4, :])
            a1 = a1 + (rows_v[r + 1, :] + rows_v[r + 5, :])
            a2 = a2 + (rows_v[r + 2, :] + rows_v[r + 6, :])
            a3 = a3 + (rows_v[r + 3, :] + rows_v[r + 7, :])
            return a0, a1, a2, a3

        a0, a1, a2, a3 = accs
        s = (a0 + a1) + (a2 + a3)

        parts = [b_vec, z, z, z]
        for k in range(D):
            parts[k % 4] = parts[k % 4] + s[k] * wt_rows[k]
        out_v[bag, :] = (parts[0] + parts[1]) + (parts[2] + parts[3])
        return carry

    lax.fori_loop(0, G, bag_body, 0)
    pltpu.sync_copy(out_v, out_hbm.at[pl.ds(obase, G), :])


def _body(text_hbm, table_hbm, wt_hbm, bias_hbm, out_hbm,
          idx0, idx1, rows0, rows1, wt_v, b_v, out_v, sem0, sem1):
    wid = lax.axis_index("s") * NC + lax.axis_index("c")
    tbase = wid * BAGS_PER_W * L
    obase = wid * BAGS_PER_W

    pltpu.sync_copy(wt_hbm, wt_v)
    pltpu.sync_copy(bias_hbm, b_v)
    wt_rows = [wt_v[k, :] for k in range(D)]
    b_vec = b_v[:]

    _fire_group(text_hbm, table_hbm, idx0, rows0, sem0, tbase)

    def outer(g2, carry):
        gA = g2 * 2
        gB = gA + 1
        _fire_group(text_hbm, table_hbm, idx1, rows1, sem1,
                    tbase + gB * IDX_PER_G)
        _drain_group(table_hbm, rows0, sem0)
        _process_group(rows0, wt_rows, b_vec, out_v, out_hbm, obase + gA * G)

        @pl.when(g2 < NG // 2 - 1)
        def _():
            _fire_group(text_hbm, table_hbm, idx0, rows0, sem0,
                        tbase + (gA + 2) * IDX_PER_G)

        _drain_group(table_hbm, rows1, sem1)
        _process_group(rows1, wt_rows, b_vec, out_v, out_hbm, obase + gB * G)
        return carry

    lax.fori_loop(0, NG // 2, outer, 0)


V = 1000000
NTC_FULL = V // 128
BT = 4
NB_MAIN = 61
TAIL_B = NB_MAIN * NW
TAIL_COLS = V - NTC_FULL * 128


def _detile_cols(tt_hbm, bufs, semi, b):
    for p in range(BT):
        pltpu.async_copy(
            tt_hbm.at[:, pl.ds((b * BT + p) * 128, 128)],
            bufs[p].at[:, pl.ds(0, 128)], semi)


def _detile_wait(tt_hbm, bufs, semi, b):
    for p in range(BT):
        pltpu.make_async_copy(
            tt_hbm.at[:, pl.ds((b * BT + p) * 128, 128)],
            bufs[p].at[:, pl.ds(0, 128)], semi,
        ).wait()


def _detile_emit(bufs, ob, ncols=128):
    iota16 = lax.iota(jnp.int32, 16)

    for p in range(BT):
        @plsc.parallel_loop(0, ncols, unroll=8)
        def _(v, p=p):
            col = plsc.load_gather(bufs[p], [iota16, jnp.full((16,), v, jnp.int32)])
            ob[pl.ds(p * 2048 + v * D, D)] = col


def _detile_body(tt_hbm, tail_hbm, out_hbm, inb, obb,
                 semi0, semi1, semi2, semi3, semo0, semo1, semo2, semo3):
    wid = lax.axis_index("s") * NC + lax.axis_index("c")
    base = wid * NB_MAIN
    semi = [semi0, semi1, semi2, semi3]
    semo = [semo0, semo1, semo2, semo3]
    slot_bufs = [[inb.at[pl.ds((s * BT + p) * 16, 16), :] for p in range(BT)]
                 for s in range(4)]
    obs = [obb.at[pl.ds(s * BT * 2048, BT * 2048)] for s in range(4)]

    for s in range(4):
        _detile_cols(tt_hbm, slot_bufs[s], semi[s], base + s)

    def ring(i, carry):
        for s in range(4):
            b = base + 4 * i + s
            _detile_wait(tt_hbm, slot_bufs[s], semi[s], b)

            @pl.when(i > 0)
            def _(s=s):
                pltpu.make_async_copy(
                    obs[s], out_hbm.at[pl.ds(0, BT * 2048)], semo[s]).wait()

            _detile_emit(slot_bufs[s], obs[s])
            pltpu.async_copy(
                obs[s], out_hbm.at[pl.ds(b * BT * 2048, BT * 2048)], semo[s])
            if s == 0:
                _detile_cols(tt_hbm, slot_bufs[s], semi[s], b + 4)
            else:
                @pl.when(i < 14)
                def _(s=s, b=b):
                    _detile_cols(tt_hbm, slot_bufs[s], semi[s], b + 4)
        return carry

    lax.fori_loop(0, 15, ring, 0)

    bL = base + NB_MAIN - 1
    _detile_wait(tt_hbm, slot_bufs[0], semi[0], bL)
    pltpu.make_async_copy(obs[0], out_hbm.at[pl.ds(0, BT * 2048)], semo[0]).wait()
    _detile_emit(slot_bufs[0], obs[0])
    pltpu.async_copy(obs[0], out_hbm.at[pl.ds(bL * BT * 2048, BT * 2048)], semo[0])
    for s in range(4):
        pltpu.make_async_copy(
            obs[s], out_hbm.at[pl.ds(0, BT * 2048)], semo[s]).wait()

    @pl.when(wid == 0)
    def _():
        _detile_cols(tt_hbm, slot_bufs[0], semi[0], TAIL_B)
        _detile_wait(tt_hbm, slot_bufs[0], semi[0], TAIL_B)
        _detile_emit(slot_bufs[0], obs[0])
        pltpu.sync_copy(obs[0], out_hbm.at[pl.ds(TAIL_B * BT * 2048, BT * 2048)])

    @pl.when(wid == 1)
    def _():
        pltpu.sync_copy(tail_hbm, obs[1].at[pl.ds(0, TAIL_COLS * D)])
        pltpu.sync_copy(obs[1].at[pl.ds(0, TAIL_COLS * D)],
                        out_hbm.at[pl.ds(NTC_FULL * 128 * D, TAIL_COLS * D)])


def _detile(table_t, tail_flat):
    run = pl.kernel(
        _detile_body,
        out_type=jax.ShapeDtypeStruct((V * D,), jnp.float32),
        mesh=plsc.VectorSubcoreMesh(core_axis_name="c", subcore_axis_name="s"),
        compiler_params=pltpu.CompilerParams(use_tc_tiling_on_sc=True,
                                             needs_layout_passes=False),
        scratch_types=(
            [pltpu.VMEM((4 * BT * 16, 129), jnp.float32),
             pltpu.VMEM((4 * BT * 2048,), jnp.float32)]
            + [pltpu.SemaphoreType.DMA for _ in range(8)]
        ),
    )
    return run(table_t, tail_flat).reshape(V, D)


@jax.jit
def kernel(text, table, W, b):
    text_flat = text.reshape(-1).astype(jnp.int32)
    table_lin = _detile(table.T, table[NTC_FULL * 128:, :].reshape(-1))
    wt = (W.T / jnp.float32(L)).astype(jnp.float32)
    run = pl.kernel(
        _body,
        out_type=jax.ShapeDtypeStruct((B, D), jnp.float32),
        mesh=plsc.VectorSubcoreMesh(core_axis_name="c", subcore_axis_name="s"),
        compiler_params=pltpu.CompilerParams(use_tc_tiling_on_sc=False),
        scratch_types=[
            pltpu.VMEM((IDX_PER_G,), jnp.int32),
            pltpu.VMEM((IDX_PER_G,), jnp.int32),
            pltpu.VMEM((IDX_PER_G, D), jnp.float32),
            pltpu.VMEM((IDX_PER_G, D), jnp.float32),
            pltpu.VMEM((D, D), jnp.float32),
            pltpu.VMEM((D,), jnp.float32),
            pltpu.VMEM((G, D), jnp.float32),
            pltpu.SemaphoreType.DMA,
            pltpu.SemaphoreType.DMA,
        ],
    )
    return run(text_flat, table_lin, wt, b.astype(jnp.float32))
